# baseline (speedup 1.0000x reference)
import functools

import jax
import jax.numpy as jnp
from jax.experimental import pallas as pl
from jax.experimental.pallas import tpu as pltpu


def _dense_kernel(a_ref, w_ref, b_ref, o_ref, *, slope):
    acc = jnp.dot(a_ref[...], w_ref[...], preferred_element_type=jnp.float32)
    acc = acc + b_ref[...][None, :]
    if slope is not None:
        acc = jnp.where(acc >= 0, acc, slope * acc)
    o_ref[...] = acc


def _dense(a, w, b, slope=None, block_rows=2000):
    n, k = a.shape
    m = w.shape[1]
    grid = (n // block_rows,)
    return pl.pallas_call(
        functools.partial(_dense_kernel, slope=slope),
        grid=grid,
        in_specs=[
            pl.BlockSpec((block_rows, k), lambda i: (i, 0)),
            pl.BlockSpec((k, m), lambda i: (0, 0)),
            pl.BlockSpec((m,), lambda i: (0,)),
        ],
        out_specs=pl.BlockSpec((block_rows, m), lambda i: (i, 0)),
        out_shape=jax.ShapeDtypeStruct((n, m), jnp.float32),
    )(a, w, b)


def _gat(h_in, src, dst, W, al, ar):
    n = h_in.shape[0]
    H, D = al.shape
    h = (h_in @ W).reshape(n, H, D)
    el = (h * al[None, :, :]).sum(-1)
    er = (h * ar[None, :, :]).sum(-1)
    e = jax.nn.leaky_relu(el[src] + er[dst], 0.2)
    emax = jax.ops.segment_max(e, dst, num_segments=n)
    ee = jnp.exp(e - emax[dst])
    esum = jax.ops.segment_sum(ee, dst, num_segments=n)
    alpha = ee / (esum[dst] + 1e-9)
    msg = h[src] * alpha[:, :, None]
    out = jax.ops.segment_sum(msg, dst, num_segments=n)
    return out.reshape(n, H * D)


def kernel(x, edge_index, W_emb, b_emb, W1, al1, ar1, Wg1, bg1, W2, al2, ar2, Wg2, bg2, W3, al3, ar3, Wg3, bg3, We, be):
    n = x.shape[0]
    src0 = edge_index[0]
    dst0 = edge_index[1]
    sl = jnp.arange(n, dtype=src0.dtype)
    src = jnp.concatenate([src0, sl])
    dst = jnp.concatenate([dst0, sl])
    node_ft = _dense(x, W_emb, b_emb)
    node_ft = _gat(node_ft, src, dst, W1, al1, ar1)
    node_ft = _dense(node_ft, Wg1, bg1, slope=0.01)
    node_ft = _gat(node_ft, src, dst, W2, al2, ar2)
    node_ft = _dense(node_ft, Wg2, bg2, slope=0.01)
    node_ft = _gat(node_ft, src, dst, W3, al3, ar3)
    node_ft = _dense(node_ft, Wg3, bg3, slope=0.01)
    edge_emb = jnp.concatenate([node_ft[src0], node_ft[dst0]], axis=1)
    edge_cls = edge_emb @ We + be
    return (node_ft, edge_emb, edge_cls)
